# baseline (speedup 1.0000x reference)
import functools

import jax
import jax.numpy as jnp
from jax import lax
from jax.experimental import pallas as pl
from jax.experimental.pallas import tpu as pltpu
from jax.experimental.pallas import tpu_sc as plsc

DIM = 64
BATCH = 16384
NC = 2
NS = 16
NW = NC * NS
BPW = BATCH // NW

_QDT = jnp.bfloat16


def _sc_gather(user_q, item_q, user, pos, neg):
    mesh = plsc.VectorSubcoreMesh(core_axis_name="c", subcore_axis_name="s")
    out_t = jax.ShapeDtypeStruct((BATCH, DIM), _QDT)

    @functools.partial(
        pl.kernel,
        mesh=mesh,
        out_type=[out_t, out_t, out_t],
        compiler_params=pltpu.CompilerParams(use_tc_tiling_on_sc=False),
        scratch_types=[
            pltpu.VMEM((BPW,), jnp.int32),
            pltpu.VMEM((BPW,), jnp.int32),
            pltpu.VMEM((BPW,), jnp.int32),
            pltpu.VMEM((BPW, DIM), _QDT),
            pltpu.VMEM((BPW, DIM), _QDT),
            pltpu.VMEM((BPW, DIM), _QDT),
            pltpu.SemaphoreType.DMA,
            pltpu.SemaphoreType.DMA,
            pltpu.SemaphoreType.DMA,
        ],
    )
    def k(ut_hbm, it_hbm, u_hbm, p_hbm, n_hbm, ue_hbm, pe_hbm, ne_hbm,
          ui_v, pi_v, ni_v, ur_v, pr_v, nr_v, su, sp, sn):
        wid = lax.axis_index("s") * NC + lax.axis_index("c")
        base = wid * BPW
        pltpu.sync_copy(u_hbm.at[pl.ds(base, BPW)], ui_v)
        pltpu.sync_copy(p_hbm.at[pl.ds(base, BPW)], pi_v)
        pltpu.sync_copy(n_hbm.at[pl.ds(base, BPW)], ni_v)
        cu = pltpu.async_copy(ut_hbm.at[ui_v], ur_v, su)
        cp = pltpu.async_copy(it_hbm.at[pi_v], pr_v, sp)
        cn = pltpu.async_copy(it_hbm.at[ni_v], nr_v, sn)
        cu.wait()
        pltpu.sync_copy(ur_v, ue_hbm.at[pl.ds(base, BPW)])
        cp.wait()
        pltpu.sync_copy(pr_v, pe_hbm.at[pl.ds(base, BPW)])
        cn.wait()
        pltpu.sync_copy(nr_v, ne_hbm.at[pl.ds(base, BPW)])

    return k(user_q, item_q, user, pos, neg)


_TC_BLK = 2048


def _tc_loss_body(u_ref, p_ref, n_ref, o_ref):
    u = u_ref[...].astype(jnp.float32)
    p = p_ref[...].astype(jnp.float32)
    n = n_ref[...].astype(jnp.float32)
    t = jnp.sum(u * (p - n), axis=1)
    part = jnp.sum(jnp.logaddexp(0.0, -t))

    @pl.when(pl.program_id(0) == 0)
    def _():
        o_ref[0] = 0.0

    o_ref[0] += part


def _tc_loss(ue, pe, ne):
    spec = pl.BlockSpec((_TC_BLK, DIM), lambda i: (i, 0))
    out = pl.pallas_call(
        _tc_loss_body,
        grid=(BATCH // _TC_BLK,),
        in_specs=[spec, spec, spec],
        out_specs=pl.BlockSpec(memory_space=pltpu.SMEM),
        out_shape=jax.ShapeDtypeStruct((1,), jnp.float32),
    )(ue, pe, ne)
    return out[0]


def kernel(user_table, item_table, user, pos, neg):
    user_q = user_table.astype(_QDT)
    item_q = item_table.astype(_QDT)
    ue, pe, ne = _sc_gather(user_q, item_q, user, pos, neg)
    return _tc_loss(ue, pe, ne)


# speedup vs baseline: 4.3699x; 4.3699x over previous
import functools

import jax
import jax.numpy as jnp
from jax import lax
from jax.experimental import pallas as pl
from jax.experimental.pallas import tpu as pltpu
from jax.experimental.pallas import tpu_sc as plsc

DIM = 64
NPAIR = DIM // 2
BATCH = 16384
NROWS = 1_000_000
PADN = 1 << 20
NC = 2
NS = 16
NW = NC * NS
BPW = BATCH // NW

_CAST_BLK = 32768

_mesh = plsc.VectorSubcoreMesh(core_axis_name="c", subcore_axis_name="s")
_sc_params = pltpu.CompilerParams(use_tc_tiling_on_sc=False)


def _cast_body(i_ref, *o_refs):
    v = i_ref[...]
    for c2 in range(NPAIR):
        lo = jax.lax.bitcast_convert_type(
            v[2 * c2, :].astype(jnp.bfloat16), jnp.uint16).astype(jnp.uint32)
        hi = jax.lax.bitcast_convert_type(
            v[2 * c2 + 1, :].astype(jnp.bfloat16), jnp.uint16).astype(jnp.uint32)
        o_refs[c2][...] = jax.lax.bitcast_convert_type(
            lo | (hi << 16), jnp.float32)


def _cast_split(tableT):
    grid = (NROWS + _CAST_BLK - 1) // _CAST_BLK
    return pl.pallas_call(
        _cast_body,
        grid=(grid,),
        in_specs=[pl.BlockSpec((DIM, _CAST_BLK), lambda k: (0, k))],
        out_specs=[pl.BlockSpec((_CAST_BLK,), lambda k: (k,))] * NPAIR,
        out_shape=[jax.ShapeDtypeStruct((PADN,), jnp.float32)] * NPAIR,
    )(tableT)


def _sc_gather_user(ufs, user):
    @functools.partial(
        pl.kernel,
        mesh=_mesh,
        out_type=jax.ShapeDtypeStruct((NPAIR * BATCH,), jnp.float32),
        compiler_params=_sc_params,
        scratch_types=[
            pltpu.VMEM((BPW,), jnp.int32),
            pltpu.VMEM((BPW,), jnp.float32),
            pltpu.VMEM((BPW,), jnp.float32),
            pltpu.SemaphoreType.DMA,
            pltpu.SemaphoreType.DMA,
        ],
    )
    def k(*refs):
        tabs = refs[:NPAIR]
        u_hbm, out_hbm = refs[NPAIR], refs[NPAIR + 1]
        ids, va, vb, sa, sb = refs[NPAIR + 2:]
        wid = lax.axis_index("s") * NC + lax.axis_index("c")
        base = wid * BPW
        pltpu.sync_copy(u_hbm.at[pl.ds(base, BPW)], ids)
        bufs = (va, vb)
        sems = (sa, sb)
        cps = [
            pltpu.async_copy(tabs[0].at[ids], va, sa),
            pltpu.async_copy(tabs[1].at[ids], vb, sb),
        ]
        for c2 in range(NPAIR):
            s = c2 % 2
            cps[s].wait()
            pltpu.sync_copy(bufs[s], out_hbm.at[pl.ds(c2 * BATCH + base, BPW)])
            if c2 + 2 < NPAIR:
                cps[s] = pltpu.async_copy(tabs[c2 + 2].at[ids], bufs[s], sems[s])

    return k(*ufs, user)


def _sc_gather_item(ifs, pos, neg):
    out_t = jax.ShapeDtypeStruct((NPAIR * BATCH,), jnp.float32)

    @functools.partial(
        pl.kernel,
        mesh=_mesh,
        out_type=[out_t, out_t],
        compiler_params=_sc_params,
        scratch_types=[
            pltpu.VMEM((BPW,), jnp.int32),
            pltpu.VMEM((BPW,), jnp.int32),
            pltpu.VMEM((BPW,), jnp.float32),
            pltpu.VMEM((BPW,), jnp.float32),
            pltpu.VMEM((BPW,), jnp.float32),
            pltpu.VMEM((BPW,), jnp.float32),
            pltpu.SemaphoreType.DMA,
            pltpu.SemaphoreType.DMA,
        ],
    )
    def k(*refs):
        tabs = refs[:NPAIR]
        p_hbm, n_hbm = refs[NPAIR], refs[NPAIR + 1]
        pg_hbm, ng_hbm = refs[NPAIR + 2], refs[NPAIR + 3]
        pids, nids, pva, pvb, nva, nvb, sa, sb = refs[NPAIR + 4:]
        wid = lax.axis_index("s") * NC + lax.axis_index("c")
        base = wid * BPW
        pltpu.sync_copy(p_hbm.at[pl.ds(base, BPW)], pids)
        pltpu.sync_copy(n_hbm.at[pl.ds(base, BPW)], nids)
        pbufs = (pva, pvb)
        nbufs = (nva, nvb)
        sems = (sa, sb)

        def fire(c2, s):
            cp = pltpu.async_copy(tabs[c2].at[pids], pbufs[s], sems[s])
            cn = pltpu.async_copy(tabs[c2].at[nids], nbufs[s], sems[s])
            return cp, cn

        cps = [fire(0, 0), fire(1, 1)]
        for c2 in range(NPAIR):
            s = c2 % 2
            for w in cps[s]:
                w.wait()
            pltpu.sync_copy(pbufs[s], pg_hbm.at[pl.ds(c2 * BATCH + base, BPW)])
            pltpu.sync_copy(nbufs[s], ng_hbm.at[pl.ds(c2 * BATCH + base, BPW)])
            if c2 + 2 < NPAIR:
                cps[s] = fire(c2 + 2, s)

    return k(*ifs, pos, neg)


def _split_pair(x):
    bits = jax.lax.bitcast_convert_type(x, jnp.uint32)
    lo = jax.lax.bitcast_convert_type(
        (bits << 16).astype(jnp.uint32), jnp.float32)
    hi = jax.lax.bitcast_convert_type(bits & jnp.uint32(0xFFFF0000), jnp.float32)
    return lo, hi


def _tc_loss_body(u_ref, p_ref, n_ref, o_ref):
    t = jnp.zeros((BATCH,), jnp.float32)
    for c2 in range(NPAIR):
        sl = pl.ds(c2 * BATCH, BATCH)
        ua, ub = _split_pair(u_ref[sl])
        pa, pb = _split_pair(p_ref[sl])
        na, nb = _split_pair(n_ref[sl])
        t = t + ua * (pa - na) + ub * (pb - nb)
    o_ref[0] = jnp.sum(jnp.logaddexp(0.0, -t))


def _tc_loss(ug, pg, ng):
    out = pl.pallas_call(
        _tc_loss_body,
        out_specs=pl.BlockSpec(memory_space=pltpu.SMEM),
        out_shape=jax.ShapeDtypeStruct((1,), jnp.float32),
    )(ug, pg, ng)
    return out[0]


def kernel(user_table, item_table, user, pos, neg):
    ufs = _cast_split(user_table.T)
    ug = _sc_gather_user(ufs, user)
    ifs = _cast_split(item_table.T)
    pg, ng = _sc_gather_item(ifs, pos, neg)
    return _tc_loss(ug, pg, ng)


# speedup vs baseline: 4.3760x; 1.0014x over previous
import functools

import jax
import jax.numpy as jnp
from jax import lax
from jax.experimental import pallas as pl
from jax.experimental.pallas import tpu as pltpu
from jax.experimental.pallas import tpu_sc as plsc

DIM = 64
NPAIR = DIM // 2
BATCH = 16384
NROWS = 1_000_000
PADN = 1 << 20
NC = 2
NS = 16
NW = NC * NS
BPW = BATCH // NW

_CAST_BLK = 32768
_SLOTS = 4

_mesh = plsc.VectorSubcoreMesh(core_axis_name="c", subcore_axis_name="s")
_sc_params = pltpu.CompilerParams(use_tc_tiling_on_sc=False)


def _cast_body(i_ref, *o_refs):
    v = i_ref[...]
    for c2 in range(NPAIR):
        lo = jax.lax.bitcast_convert_type(
            v[2 * c2, :].astype(jnp.bfloat16), jnp.uint16).astype(jnp.uint32)
        hi = jax.lax.bitcast_convert_type(
            v[2 * c2 + 1, :].astype(jnp.bfloat16), jnp.uint16).astype(jnp.uint32)
        o_refs[c2][...] = jax.lax.bitcast_convert_type(
            lo | (hi << 16), jnp.float32)


def _cast_split(tableT):
    grid = (NROWS + _CAST_BLK - 1) // _CAST_BLK
    return pl.pallas_call(
        _cast_body,
        grid=(grid,),
        in_specs=[pl.BlockSpec((DIM, _CAST_BLK), lambda k: (0, k))],
        out_specs=[pl.BlockSpec((_CAST_BLK,), lambda k: (k,))] * NPAIR,
        out_shape=[jax.ShapeDtypeStruct((PADN,), jnp.float32)] * NPAIR,
    )(tableT)


def _sc_gather_user(ufs, user):
    @functools.partial(
        pl.kernel,
        mesh=_mesh,
        out_type=jax.ShapeDtypeStruct((NPAIR * BATCH,), jnp.float32),
        compiler_params=_sc_params,
        scratch_types=(
            [pltpu.VMEM((BPW,), jnp.int32)]
            + [pltpu.VMEM((BPW,), jnp.float32)] * _SLOTS
            + [pltpu.SemaphoreType.DMA] * (_SLOTS + 1)
        ),
    )
    def k(*refs):
        tabs = refs[:NPAIR]
        u_hbm, out_hbm = refs[NPAIR], refs[NPAIR + 1]
        ids = refs[NPAIR + 2]
        bufs = refs[NPAIR + 3:NPAIR + 3 + _SLOTS]
        sems = refs[NPAIR + 3 + _SLOTS:NPAIR + 3 + 2 * _SLOTS]
        oss = refs[NPAIR + 3 + 2 * _SLOTS]
        wid = lax.axis_index("s") * NC + lax.axis_index("c")
        base = wid * BPW
        pltpu.sync_copy(u_hbm.at[pl.ds(base, BPW)], ids)
        cps = [pltpu.async_copy(tabs[s].at[ids], bufs[s], sems[s])
               for s in range(_SLOTS)]
        outs = [None] * NPAIR
        for c2 in range(NPAIR):
            s = c2 % _SLOTS
            cps[s].wait()
            outs[c2] = pltpu.async_copy(
                bufs[s], out_hbm.at[pl.ds(c2 * BATCH + base, BPW)], oss)
            if c2 >= 1 and c2 - 1 + _SLOTS < NPAIR:
                outs[c2 - 1].wait()
                sp = (c2 - 1) % _SLOTS
                cps[sp] = pltpu.async_copy(
                    tabs[c2 - 1 + _SLOTS].at[ids], bufs[sp], sems[sp])
        for c2 in range(NPAIR - _SLOTS, NPAIR):
            outs[c2].wait()

    return k(*ufs, user)


def _sc_gather_item(ifs, pos, neg):
    out_t = jax.ShapeDtypeStruct((NPAIR * BATCH,), jnp.float32)

    @functools.partial(
        pl.kernel,
        mesh=_mesh,
        out_type=[out_t, out_t],
        compiler_params=_sc_params,
        scratch_types=(
            [pltpu.VMEM((BPW,), jnp.int32)] * 2
            + [pltpu.VMEM((BPW,), jnp.float32)] * (2 * _SLOTS)
            + [pltpu.SemaphoreType.DMA] * (_SLOTS + 1)
        ),
    )
    def k(*refs):
        tabs = refs[:NPAIR]
        p_hbm, n_hbm = refs[NPAIR], refs[NPAIR + 1]
        pg_hbm, ng_hbm = refs[NPAIR + 2], refs[NPAIR + 3]
        pids, nids = refs[NPAIR + 4], refs[NPAIR + 5]
        pbufs = refs[NPAIR + 6:NPAIR + 6 + _SLOTS]
        nbufs = refs[NPAIR + 6 + _SLOTS:NPAIR + 6 + 2 * _SLOTS]
        sems = refs[NPAIR + 6 + 2 * _SLOTS:NPAIR + 6 + 3 * _SLOTS]
        oss = refs[NPAIR + 6 + 3 * _SLOTS]
        wid = lax.axis_index("s") * NC + lax.axis_index("c")
        base = wid * BPW
        pltpu.sync_copy(p_hbm.at[pl.ds(base, BPW)], pids)
        pltpu.sync_copy(n_hbm.at[pl.ds(base, BPW)], nids)

        def fire(c2, s):
            cp = pltpu.async_copy(tabs[c2].at[pids], pbufs[s], sems[s])
            cn = pltpu.async_copy(tabs[c2].at[nids], nbufs[s], sems[s])
            return cp, cn

        cps = [fire(s, s) for s in range(_SLOTS)]
        outs = [None] * NPAIR
        for c2 in range(NPAIR):
            s = c2 % _SLOTS
            for w in cps[s]:
                w.wait()
            op = pltpu.async_copy(
                pbufs[s], pg_hbm.at[pl.ds(c2 * BATCH + base, BPW)], oss)
            on = pltpu.async_copy(
                nbufs[s], ng_hbm.at[pl.ds(c2 * BATCH + base, BPW)], oss)
            outs[c2] = (op, on)
            if c2 >= 1 and c2 - 1 + _SLOTS < NPAIR:
                for w in outs[c2 - 1]:
                    w.wait()
                sp = (c2 - 1) % _SLOTS
                cps[sp] = fire(c2 - 1 + _SLOTS, sp)
        for c2 in range(NPAIR - _SLOTS, NPAIR):
            for w in outs[c2]:
                w.wait()

    return k(*ifs, pos, neg)


def _split_pair(x):
    bits = jax.lax.bitcast_convert_type(x, jnp.uint32)
    lo = jax.lax.bitcast_convert_type(
        (bits << 16).astype(jnp.uint32), jnp.float32)
    hi = jax.lax.bitcast_convert_type(bits & jnp.uint32(0xFFFF0000), jnp.float32)
    return lo, hi


def _tc_loss_body(u_ref, p_ref, n_ref, o_ref):
    t = jnp.zeros((BATCH,), jnp.float32)
    for c2 in range(NPAIR):
        sl = pl.ds(c2 * BATCH, BATCH)
        ua, ub = _split_pair(u_ref[sl])
        pa, pb = _split_pair(p_ref[sl])
        na, nb = _split_pair(n_ref[sl])
        t = t + ua * (pa - na) + ub * (pb - nb)
    o_ref[0] = jnp.sum(jnp.logaddexp(0.0, -t))


def _tc_loss(ug, pg, ng):
    out = pl.pallas_call(
        _tc_loss_body,
        out_specs=pl.BlockSpec(memory_space=pltpu.SMEM),
        out_shape=jax.ShapeDtypeStruct((1,), jnp.float32),
    )(ug, pg, ng)
    return out[0]


def kernel(user_table, item_table, user, pos, neg):
    ifs = _cast_split(item_table.T)
    pg, ng = _sc_gather_item(ifs, pos, neg)
    ufs = _cast_split(user_table.T)
    ug = _sc_gather_user(ufs, user)
    return _tc_loss(ug, pg, ng)
